# baseline (speedup 1.0000x reference)
import functools

import jax
import jax.numpy as jnp
from jax import lax
from jax.experimental import pallas as pl
from jax.experimental.pallas import tpu as pltpu
from jax.experimental.pallas import tpu_sc as plsc

N_NODES = 10000
N_EDGES = 320000
D = 128
WOUT_PAD = 64
N_CLASS = 41

NC = 2
NS = 16
NW = NC * NS

NP = 10240
ROWS_PT = NP // NS
EPT = N_EDGES // NW
CHUNK = 80
NCHUNK = EPT // CHUNK
NBUF = 4
GDEPTH = 2
NIB = 8
UNROLL = 8


def _make_edge_scatter(W, dtype=jnp.float32, tc_tiling=True):
  mesh = plsc.VectorSubcoreMesh(core_axis_name="c", subcore_axis_name="s")

  rows_t = [pltpu.VMEM((CHUNK, W), dtype) for _ in range(NBUF)]
  sidx_t = [pltpu.VMEM((CHUNK,), jnp.int32) for _ in range(NIB)]
  didx_t = [pltpu.VMEM((CHUNK,), jnp.int32) for _ in range(NIB)]
  sem_t = [pltpu.SemaphoreType.DMA for _ in range(2 * NBUF + NIB)]

  @functools.partial(
      pl.kernel,
      mesh=mesh,
      compiler_params=pltpu.CompilerParams(
          use_tc_tiling_on_sc=None if tc_tiling else False),
      out_type=jax.ShapeDtypeStruct((NC, NP, W), dtype),
      scratch_types=[
          pltpu.VMEM_SHARED((NP, W), dtype),
      ] + rows_t + sidx_t + didx_t + sem_t,
  )
  def scatter_kernel(support, src3d, dst3d, out, agg, *bufs):
    rows = bufs[:NBUF]
    sidx = bufs[NBUF:NBUF + NIB]
    didx = bufs[NBUF + NIB:NBUF + 2 * NIB]
    gsem = bufs[NBUF + 2 * NIB:2 * NBUF + 2 * NIB]
    ssem = bufs[2 * NBUF + 2 * NIB:3 * NBUF + 2 * NIB]
    isem = bufs[3 * NBUF + 2 * NIB:3 * NBUF + 3 * NIB]
    cid = lax.axis_index("c")
    sid = lax.axis_index("s")
    wid = cid * NS + sid
    rbase = sid * ROWS_PT

    ebase = wid * EPT

    def i_start(j, b):
      e0 = ebase + j * CHUNK
      pltpu.async_copy(src3d.at[pl.ds(e0, CHUNK)], sidx[b], isem[b])
      pltpu.async_copy(dst3d.at[pl.ds(e0, CHUNK)], didx[b], isem[b])

    def i_wait(j, b):
      e0 = ebase + j * CHUNK
      pltpu.make_async_copy(src3d.at[pl.ds(e0, CHUNK)], sidx[b],
                            isem[b]).wait()
      pltpu.make_async_copy(dst3d.at[pl.ds(e0, CHUNK)], didx[b],
                            isem[b]).wait()

    def g_start(j, b, ib):
      pltpu.async_copy(support.at[sidx[ib]], rows[b], gsem[b])

    def g_wait(j, b, ib):
      pltpu.make_async_copy(support.at[sidx[ib]], rows[b], gsem[b]).wait()

    def s_start(j, b, ib):
      pltpu.async_copy(rows[b], agg.at[didx[ib]], ssem[b], add=True)

    def s_wait(j, b, ib):
      pltpu.make_async_copy(rows[b], agg.at[didx[ib]], ssem[b]).wait()

    def slot(j, b, ib, swait, gstart, istart):
      g_wait(j, b, ib)
      s_start(j, b, ib)
      if swait:
        s_wait(j - GDEPTH, (b + GDEPTH) % NBUF, (ib - GDEPTH) % NIB)
      if gstart:
        i_wait(j + GDEPTH, (ib + GDEPTH) % NIB)
        g_start(j + GDEPTH, (b + GDEPTH) % NBUF, (ib + GDEPTH) % NIB)
      if istart:
        i_start(j + GDEPTH + 1, (ib + GDEPTH + 1) % NIB)

    for j in range(GDEPTH + 1):
      i_start(j, j % NIB)
    for j in range(GDEPTH):
      i_wait(j, j % NIB)
      g_start(j, j % NBUF, j % NIB)

    zbufs = list(range(GDEPTH, NBUF))
    if dtype == jnp.float32:
      zvec = jnp.zeros((16,), dtype)

      def zfill_body(r, _):
        for q in range(W // 16):
          for zb in zbufs:
            rows[zb][r, pl.ds(q * 16, 16)] = zvec
        return _
      lax.fori_loop(0, CHUNK, zfill_body, None)
    else:
      zvec = jnp.zeros((2, 16), dtype)

      def zfill_body(i, _):
        for q in range(W // 16):
          for zb in zbufs:
            rows[zb][pl.ds(2 * i, 2), pl.ds(q * 16, 16)] = zvec
        return _
      lax.fori_loop(0, CHUNK // 2, zfill_body, None)
    for i in range(ROWS_PT // CHUNK):
      b = zbufs[i % len(zbufs)]
      pltpu.async_copy(rows[b], agg.at[pl.ds(rbase + i * CHUNK, CHUNK)],
                       ssem[b])
    for i in range(ROWS_PT // CHUNK):
      b = zbufs[i % len(zbufs)]
      pltpu.make_async_copy(rows[b], agg.at[pl.ds(rbase + i * CHUNK, CHUNK)],
                            ssem[b]).wait()
    plsc.subcore_barrier()

    for j in range(UNROLL):
      slot(j, j % NBUF, j % NIB, swait=(j >= GDEPTH), gstart=True,
           istart=True)

    def ring_body(g, _):
      j0 = g * UNROLL
      for k in range(UNROLL):
        slot(j0 + k, k % NBUF, k % NIB, swait=True, gstart=True,
             istart=True)
      return _
    _K = (NCHUNK - UNROLL - 1) // UNROLL
    lax.fori_loop(1, _K, ring_body, None)

    for j in range(_K * UNROLL, NCHUNK):
      slot(j, j % NBUF, j % NIB, swait=True,
           gstart=(j + GDEPTH < NCHUNK),
           istart=(j + GDEPTH + 1 < NCHUNK))
    for j in range(NCHUNK - GDEPTH, NCHUNK):
      s_wait(j, j % NBUF, j % NIB)
    plsc.subcore_barrier()

    pltpu.sync_copy(agg.at[pl.ds(rbase, ROWS_PT)],
                    out.at[cid].at[pl.ds(rbase, ROWS_PT)])

  return scatter_kernel


_scatter128 = _make_edge_scatter(D, jnp.float32, tc_tiling=True)

_BR = 1024


def _mm_body(x_ref, w_ref, o_ref):
  o_ref[...] = jnp.dot(x_ref[...], w_ref[...],
                       preferred_element_type=jnp.float32
                       ).astype(o_ref.dtype)


def _combine_mm_body(p_ref, b_ref, w_ref, o_ref):
  p = p_ref[0].astype(jnp.float32) + p_ref[1].astype(jnp.float32)
  h = jnp.maximum(p + b_ref[...], 0.0)
  o_ref[...] = jnp.dot(h, w_ref[...], preferred_element_type=jnp.float32
                       ).astype(o_ref.dtype)


def _combine_body(p_ref, b_ref, o_ref):
  p = p_ref[0].astype(jnp.float32) + p_ref[1].astype(jnp.float32)
  o_ref[...] = jnp.maximum(p + b_ref[...], 0.0).astype(o_ref.dtype)


def _final_body(p_ref, w_ref, b_ref, o_ref):
  agg = (p_ref[0].astype(jnp.float32)
         + p_ref[1].astype(jnp.float32))
  v = jnp.dot(agg, w_ref[...],
              preferred_element_type=jnp.float32) + b_ref[...]
  col = lax.broadcasted_iota(jnp.int32, v.shape, 1)
  valid = col < N_CLASS
  vm = jnp.where(valid, v, -jnp.inf)
  m = jnp.max(vm, axis=1, keepdims=True)
  ex = jnp.where(valid, jnp.exp(v - m), 0.0)
  lse = jnp.log(jnp.sum(ex, axis=1, keepdims=True)) + m
  o_ref[...] = v - lse


def _tc_matmul(x, w, out_dtype=jnp.float32):
  n, k = x.shape
  kw, m = w.shape
  return pl.pallas_call(
      _mm_body,
      grid=(n // _BR,),
      in_specs=[pl.BlockSpec((_BR, k), lambda i: (i, 0)),
                pl.BlockSpec((kw, m), lambda i: (0, 0))],
      out_specs=pl.BlockSpec((_BR, m), lambda i: (i, 0)),
      out_shape=jax.ShapeDtypeStruct((n, m), out_dtype),
  )(x, w)


def _tc_combine_mm(p, b, w, out_dtype=jnp.float32):
  _, n, k = p.shape
  kw, m = w.shape
  return pl.pallas_call(
      _combine_mm_body,
      grid=(n // _BR,),
      in_specs=[pl.BlockSpec((NC, _BR, k), lambda i: (0, i, 0)),
                pl.BlockSpec((1, k), lambda i: (0, 0)),
                pl.BlockSpec((kw, m), lambda i: (0, 0))],
      out_specs=pl.BlockSpec((_BR, m), lambda i: (i, 0)),
      out_shape=jax.ShapeDtypeStruct((n, m), out_dtype),
  )(p, b, w)


def _tc_combine(p, b, out_dtype=jnp.float32):
  _, n, k = p.shape
  return pl.pallas_call(
      _combine_body,
      grid=(n // _BR,),
      in_specs=[pl.BlockSpec((NC, _BR, k), lambda i: (0, i, 0)),
                pl.BlockSpec((1, k), lambda i: (0, 0))],
      out_specs=pl.BlockSpec((_BR, k), lambda i: (i, 0)),
      out_shape=jax.ShapeDtypeStruct((n, k), out_dtype),
  )(p, b)


def _tc_final(p, w, b):
  _, n, k = p.shape
  kw, m = w.shape
  return pl.pallas_call(
      _final_body,
      grid=(n // _BR,),
      in_specs=[pl.BlockSpec((NC, _BR, k), lambda i: (0, i, 0)),
                pl.BlockSpec((kw, m), lambda i: (0, 0)),
                pl.BlockSpec((1, m), lambda i: (0, 0))],
      out_specs=pl.BlockSpec((_BR, m), lambda i: (i, 0)),
      out_shape=jax.ShapeDtypeStruct((n, m), jnp.float32),
  )(p, w, b)


def kernel(x, edge_index, W0, b0, W1, b1, Wout, bout):
  src = edge_index[0].astype(jnp.int32)
  dst = edge_index[1].astype(jnp.int32)
  xp = jnp.pad(x, ((0, NP - N_NODES), (0, 0)))
  wout_p = jnp.pad(Wout, ((0, 0), (0, WOUT_PAD - N_CLASS)))
  bout_p = jnp.pad(bout, (0, WOUT_PAD - N_CLASS)).reshape(1, WOUT_PAD)
  b0r = b0.reshape(1, D)
  b1r = b1.reshape(1, D)

  bf = jnp.float32
  support0 = _tc_matmul(xp, W0, bf)
  p0 = _scatter128(support0, src, dst)
  support1 = _tc_combine_mm(p0, b0r, W1, bf)
  p1 = _scatter128(support1, src, dst)
  h2 = _tc_combine(p1, b1r, bf)
  p2 = _scatter128(h2, src, dst)
  pred = _tc_final(p2, wout_p, bout_p)
  return pred[:N_NODES, :N_CLASS]


# speedup vs baseline: 1.0250x; 1.0250x over previous
import functools

import jax
import jax.numpy as jnp
from jax import lax
from jax.experimental import pallas as pl
from jax.experimental.pallas import tpu as pltpu
from jax.experimental.pallas import tpu_sc as plsc

N_NODES = 10000
N_EDGES = 320000
D = 128
WOUT_PAD = 64
N_CLASS = 41

NC = 2
NS = 16
NW = NC * NS

NP = 10240
ROWS_PT = NP // NS
EPT = N_EDGES // NW
CHUNK = 80
NCHUNK = EPT // CHUNK
NBUF = 4
GDEPTH = 2
NIB = 8
UNROLL = 8


def _make_edge_scatter(W, dtype=jnp.float32, tc_tiling=True):
  mesh = plsc.VectorSubcoreMesh(core_axis_name="c", subcore_axis_name="s")

  rows_t = [pltpu.VMEM((CHUNK, W), dtype) for _ in range(NBUF)]
  sidx_t = [pltpu.VMEM((CHUNK,), jnp.int32) for _ in range(NIB)]
  didx_t = [pltpu.VMEM((CHUNK,), jnp.int32) for _ in range(NIB)]
  sem_t = [pltpu.SemaphoreType.DMA for _ in range(2 * NBUF + NIB)]

  @functools.partial(
      pl.kernel,
      mesh=mesh,
      compiler_params=pltpu.CompilerParams(
          use_tc_tiling_on_sc=None if tc_tiling else False),
      out_type=jax.ShapeDtypeStruct((NC, NP, W), dtype),
      scratch_types=[
          pltpu.VMEM_SHARED((NP, W), dtype),
      ] + rows_t + sidx_t + didx_t + sem_t,
  )
  def scatter_kernel(support, src3d, dst3d, out, agg, *bufs):
    rows = bufs[:NBUF]
    sidx = bufs[NBUF:NBUF + NIB]
    didx = bufs[NBUF + NIB:NBUF + 2 * NIB]
    gsem = bufs[NBUF + 2 * NIB:2 * NBUF + 2 * NIB]
    ssem = bufs[2 * NBUF + 2 * NIB:3 * NBUF + 2 * NIB]
    isem = bufs[3 * NBUF + 2 * NIB:3 * NBUF + 3 * NIB]
    cid = lax.axis_index("c")
    sid = lax.axis_index("s")
    wid = cid * NS + sid
    rbase = sid * ROWS_PT

    ebase = wid * EPT

    def i_start(j, b):
      e0 = ebase + j * CHUNK
      pltpu.async_copy(src3d.at[pl.ds(e0, CHUNK)], sidx[b], isem[b])
      pltpu.async_copy(dst3d.at[pl.ds(e0, CHUNK)], didx[b], isem[b])

    def i_wait(j, b):
      e0 = ebase + j * CHUNK
      pltpu.make_async_copy(src3d.at[pl.ds(e0, CHUNK)], sidx[b],
                            isem[b]).wait()
      pltpu.make_async_copy(dst3d.at[pl.ds(e0, CHUNK)], didx[b],
                            isem[b]).wait()

    def g_start(j, b, ib):
      pltpu.async_copy(support.at[sidx[ib]], rows[b], gsem[b])

    def g_wait(j, b, ib):
      pltpu.make_async_copy(support.at[sidx[ib]], rows[b], gsem[b]).wait()

    def s_start(j, b, ib):
      pltpu.async_copy(rows[b], agg.at[didx[ib]], ssem[b], add=True)

    def s_wait(j, b, ib):
      pltpu.make_async_copy(rows[b], agg.at[didx[ib]], ssem[b]).wait()

    def slot(j, b, ib, swait, gstart, istart):
      g_wait(j, b, ib)
      s_start(j, b, ib)
      if swait:
        s_wait(j - GDEPTH, (b + GDEPTH) % NBUF, (ib - GDEPTH) % NIB)
      if gstart:
        i_wait(j + GDEPTH, (ib + GDEPTH) % NIB)
        g_start(j + GDEPTH, (b + GDEPTH) % NBUF, (ib + GDEPTH) % NIB)
      if istart:
        i_start(j + GDEPTH + 1, (ib + GDEPTH + 1) % NIB)

    for j in range(GDEPTH + 1):
      i_start(j, j % NIB)
    for j in range(GDEPTH):
      i_wait(j, j % NIB)
      g_start(j, j % NBUF, j % NIB)

    zbufs = list(range(GDEPTH, NBUF))
    if dtype == jnp.float32:
      zvec = jnp.zeros((16,), dtype)

      def zfill_body(r, _):
        for q in range(W // 16):
          for zb in zbufs:
            rows[zb][r, pl.ds(q * 16, 16)] = zvec
        return _
      lax.fori_loop(0, CHUNK, zfill_body, None)
    else:
      zvec = jnp.zeros((2, 16), dtype)

      def zfill_body(i, _):
        for q in range(W // 16):
          for zb in zbufs:
            rows[zb][pl.ds(2 * i, 2), pl.ds(q * 16, 16)] = zvec
        return _
      lax.fori_loop(0, CHUNK // 2, zfill_body, None)
    for i in range(ROWS_PT // CHUNK):
      b = zbufs[i % len(zbufs)]
      pltpu.async_copy(rows[b], agg.at[pl.ds(rbase + i * CHUNK, CHUNK)],
                       ssem[b])
    for i in range(ROWS_PT // CHUNK):
      b = zbufs[i % len(zbufs)]
      pltpu.make_async_copy(rows[b], agg.at[pl.ds(rbase + i * CHUNK, CHUNK)],
                            ssem[b]).wait()
    plsc.subcore_barrier()

    for j in range(UNROLL):
      slot(j, j % NBUF, j % NIB, swait=(j >= GDEPTH), gstart=True,
           istart=True)

    def ring_body(g, _):
      j0 = g * UNROLL
      for k in range(UNROLL):
        slot(j0 + k, k % NBUF, k % NIB, swait=True, gstart=True,
             istart=True)
      return _
    _K = (NCHUNK - UNROLL - 1) // UNROLL
    lax.fori_loop(1, _K, ring_body, None)

    for j in range(_K * UNROLL, NCHUNK):
      slot(j, j % NBUF, j % NIB, swait=True,
           gstart=(j + GDEPTH < NCHUNK),
           istart=(j + GDEPTH + 1 < NCHUNK))
    for j in range(NCHUNK - GDEPTH, NCHUNK):
      s_wait(j, j % NBUF, j % NIB)
    plsc.subcore_barrier()

    pltpu.sync_copy(agg.at[pl.ds(rbase, ROWS_PT)],
                    out.at[cid].at[pl.ds(rbase, ROWS_PT)])

  return scatter_kernel


_scatter128 = _make_edge_scatter(D, jnp.float32, tc_tiling=True)
_scatter64 = _make_edge_scatter(WOUT_PAD, jnp.float32, tc_tiling=False)

_BR = 1024


def _mm_body(x_ref, w_ref, o_ref):
  o_ref[...] = jnp.dot(x_ref[...], w_ref[...],
                       preferred_element_type=jnp.float32
                       ).astype(o_ref.dtype)


def _combine_mm_body(p_ref, b_ref, w_ref, o_ref):
  p = p_ref[0].astype(jnp.float32) + p_ref[1].astype(jnp.float32)
  h = jnp.maximum(p + b_ref[...], 0.0)
  o_ref[...] = jnp.dot(h, w_ref[...], preferred_element_type=jnp.float32
                       ).astype(o_ref.dtype)


def _combine_body(p_ref, b_ref, o_ref):
  p = p_ref[0].astype(jnp.float32) + p_ref[1].astype(jnp.float32)
  o_ref[...] = jnp.maximum(p + b_ref[...], 0.0).astype(o_ref.dtype)


def _final_body(p_ref, b_ref, o_ref):
  v = (p_ref[0].astype(jnp.float32)
       + p_ref[1].astype(jnp.float32)) + b_ref[...]
  col = lax.broadcasted_iota(jnp.int32, v.shape, 1)
  valid = col < N_CLASS
  vm = jnp.where(valid, v, -jnp.inf)
  m = jnp.max(vm, axis=1, keepdims=True)
  ex = jnp.where(valid, jnp.exp(v - m), 0.0)
  lse = jnp.log(jnp.sum(ex, axis=1, keepdims=True)) + m
  o_ref[...] = v - lse


def _tc_matmul(x, w, out_dtype=jnp.float32):
  n, k = x.shape
  kw, m = w.shape
  return pl.pallas_call(
      _mm_body,
      grid=(n // _BR,),
      in_specs=[pl.BlockSpec((_BR, k), lambda i: (i, 0)),
                pl.BlockSpec((kw, m), lambda i: (0, 0))],
      out_specs=pl.BlockSpec((_BR, m), lambda i: (i, 0)),
      out_shape=jax.ShapeDtypeStruct((n, m), out_dtype),
  )(x, w)


def _tc_combine_mm(p, b, w, out_dtype=jnp.float32):
  _, n, k = p.shape
  kw, m = w.shape
  return pl.pallas_call(
      _combine_mm_body,
      grid=(n // _BR,),
      in_specs=[pl.BlockSpec((NC, _BR, k), lambda i: (0, i, 0)),
                pl.BlockSpec((1, k), lambda i: (0, 0)),
                pl.BlockSpec((kw, m), lambda i: (0, 0))],
      out_specs=pl.BlockSpec((_BR, m), lambda i: (i, 0)),
      out_shape=jax.ShapeDtypeStruct((n, m), out_dtype),
  )(p, b, w)


def _tc_combine(p, b, out_dtype=jnp.float32):
  _, n, k = p.shape
  return pl.pallas_call(
      _combine_body,
      grid=(n // _BR,),
      in_specs=[pl.BlockSpec((NC, _BR, k), lambda i: (0, i, 0)),
                pl.BlockSpec((1, k), lambda i: (0, 0))],
      out_specs=pl.BlockSpec((_BR, k), lambda i: (i, 0)),
      out_shape=jax.ShapeDtypeStruct((n, k), out_dtype),
  )(p, b)


def _tc_final(p, b):
  _, n, m = p.shape
  return pl.pallas_call(
      _final_body,
      grid=(n // _BR,),
      in_specs=[pl.BlockSpec((NC, _BR, m), lambda i: (0, i, 0)),
                pl.BlockSpec((1, m), lambda i: (0, 0))],
      out_specs=pl.BlockSpec((_BR, m), lambda i: (i, 0)),
      out_shape=jax.ShapeDtypeStruct((n, m), jnp.float32),
  )(p, b)


def kernel(x, edge_index, W0, b0, W1, b1, Wout, bout):
  src = edge_index[0].astype(jnp.int32)
  dst = edge_index[1].astype(jnp.int32)
  xp = jnp.pad(x, ((0, NP - N_NODES), (0, 0)))
  wout_p = jnp.pad(Wout, ((0, 0), (0, WOUT_PAD - N_CLASS)))
  bout_p = jnp.pad(bout, (0, WOUT_PAD - N_CLASS)).reshape(1, WOUT_PAD)
  b0r = b0.reshape(1, D)
  b1r = b1.reshape(1, D)

  bf = jnp.float32
  support0 = _tc_matmul(xp, W0, bf)
  p0 = _scatter128(support0, src, dst)
  support1 = _tc_combine_mm(p0, b0r, W1, bf)
  p1 = _scatter128(support1, src, dst)
  support2 = _tc_combine_mm(p1, b1r, wout_p)
  p2 = _scatter64(support2, src, dst)
  pred = _tc_final(p2, bout_p)
  return pred[:N_NODES, :N_CLASS]


# speedup vs baseline: 1.0653x; 1.0393x over previous
import functools

import jax
import jax.numpy as jnp
from jax import lax
from jax.experimental import pallas as pl
from jax.experimental.pallas import tpu as pltpu
from jax.experimental.pallas import tpu_sc as plsc

N_NODES = 10000
N_EDGES = 320000
D = 128
WOUT_PAD = 64
N_CLASS = 41

NC = 2
NS = 16
NW = NC * NS

NP = 10240
ROWS_PT = NP // NS
EPT = N_EDGES // NW
CHUNK = 80
NCHUNK = EPT // CHUNK
NBUF = 4
GDEPTH = 2
NIB = 8
UNROLL = 8


def _make_edge_scatter(W, dtype=jnp.float32, tc_tiling=True):
  mesh = plsc.VectorSubcoreMesh(core_axis_name="c", subcore_axis_name="s")

  rows_t = [pltpu.VMEM((CHUNK, W), dtype) for _ in range(NBUF)]
  sidx_t = [pltpu.VMEM((CHUNK,), jnp.int32) for _ in range(NIB)]
  didx_t = [pltpu.VMEM((CHUNK,), jnp.int32) for _ in range(NIB)]
  sem_t = [pltpu.SemaphoreType.DMA for _ in range(2 * NBUF + NIB)]

  @functools.partial(
      pl.kernel,
      mesh=mesh,
      compiler_params=pltpu.CompilerParams(
          use_tc_tiling_on_sc=None if tc_tiling else False),
      out_type=jax.ShapeDtypeStruct((NC, NP, W), dtype),
      scratch_types=[
          pltpu.VMEM_SHARED((NP, W), dtype),
      ] + rows_t + sidx_t + didx_t + sem_t,
  )
  def scatter_kernel(support, edge_flat, out, agg, *bufs):
    rows = bufs[:NBUF]
    sidx = bufs[NBUF:NBUF + NIB]
    didx = bufs[NBUF + NIB:NBUF + 2 * NIB]
    gsem = bufs[NBUF + 2 * NIB:2 * NBUF + 2 * NIB]
    ssem = bufs[2 * NBUF + 2 * NIB:3 * NBUF + 2 * NIB]
    isem = bufs[3 * NBUF + 2 * NIB:3 * NBUF + 3 * NIB]
    cid = lax.axis_index("c")
    sid = lax.axis_index("s")
    wid = cid * NS + sid
    rbase = sid * ROWS_PT

    ebase = wid * EPT

    def i_start(j, b):
      e0 = ebase + j * CHUNK
      pltpu.async_copy(edge_flat.at[pl.ds(e0, CHUNK)], sidx[b], isem[b])
      pltpu.async_copy(edge_flat.at[pl.ds(N_EDGES + e0, CHUNK)], didx[b],
                       isem[b])

    def i_wait(j, b):
      e0 = ebase + j * CHUNK
      pltpu.make_async_copy(edge_flat.at[pl.ds(e0, CHUNK)], sidx[b],
                            isem[b]).wait()
      pltpu.make_async_copy(edge_flat.at[pl.ds(N_EDGES + e0, CHUNK)],
                            didx[b], isem[b]).wait()

    def g_start(j, b, ib):
      pltpu.async_copy(support.at[sidx[ib]], rows[b], gsem[b])

    def g_wait(j, b, ib):
      pltpu.make_async_copy(support.at[sidx[ib]], rows[b], gsem[b]).wait()

    def s_start(j, b, ib):
      pltpu.async_copy(rows[b], agg.at[didx[ib]], ssem[b], add=True)

    def s_wait(j, b, ib):
      pltpu.make_async_copy(rows[b], agg.at[didx[ib]], ssem[b]).wait()

    def slot(j, b, ib, swait, gstart, istart):
      g_wait(j, b, ib)
      s_start(j, b, ib)
      if swait:
        s_wait(j - GDEPTH, (b + GDEPTH) % NBUF, (ib - GDEPTH) % NIB)
      if gstart:
        i_wait(j + GDEPTH, (ib + GDEPTH) % NIB)
        g_start(j + GDEPTH, (b + GDEPTH) % NBUF, (ib + GDEPTH) % NIB)
      if istart:
        i_start(j + GDEPTH + 1, (ib + GDEPTH + 1) % NIB)

    for j in range(GDEPTH + 1):
      i_start(j, j % NIB)
    for j in range(GDEPTH):
      i_wait(j, j % NIB)
      g_start(j, j % NBUF, j % NIB)

    zbufs = list(range(GDEPTH, NBUF))
    if dtype == jnp.float32:
      zvec = jnp.zeros((16,), dtype)

      def zfill_body(r, _):
        for q in range(W // 16):
          for zb in zbufs:
            rows[zb][r, pl.ds(q * 16, 16)] = zvec
        return _
      lax.fori_loop(0, CHUNK, zfill_body, None)
    else:
      zvec = jnp.zeros((2, 16), dtype)

      def zfill_body(i, _):
        for q in range(W // 16):
          for zb in zbufs:
            rows[zb][pl.ds(2 * i, 2), pl.ds(q * 16, 16)] = zvec
        return _
      lax.fori_loop(0, CHUNK // 2, zfill_body, None)
    for i in range(ROWS_PT // CHUNK):
      b = zbufs[i % len(zbufs)]
      pltpu.async_copy(rows[b], agg.at[pl.ds(rbase + i * CHUNK, CHUNK)],
                       ssem[b])
    for i in range(ROWS_PT // CHUNK):
      b = zbufs[i % len(zbufs)]
      pltpu.make_async_copy(rows[b], agg.at[pl.ds(rbase + i * CHUNK, CHUNK)],
                            ssem[b]).wait()
    plsc.subcore_barrier()

    for j in range(UNROLL):
      slot(j, j % NBUF, j % NIB, swait=(j >= GDEPTH), gstart=True,
           istart=True)

    def ring_body(g, _):
      j0 = g * UNROLL
      for k in range(UNROLL):
        slot(j0 + k, k % NBUF, k % NIB, swait=True, gstart=True,
             istart=True)
      return _
    _K = (NCHUNK - UNROLL - 1) // UNROLL
    lax.fori_loop(1, _K, ring_body, None)

    for j in range(_K * UNROLL, NCHUNK):
      slot(j, j % NBUF, j % NIB, swait=True,
           gstart=(j + GDEPTH < NCHUNK),
           istart=(j + GDEPTH + 1 < NCHUNK))
    for j in range(NCHUNK - GDEPTH, NCHUNK):
      s_wait(j, j % NBUF, j % NIB)
    plsc.subcore_barrier()

    pltpu.sync_copy(agg.at[pl.ds(rbase, ROWS_PT)],
                    out.at[cid].at[pl.ds(rbase, ROWS_PT)])

  return scatter_kernel


_scatter128 = _make_edge_scatter(D, jnp.float32, tc_tiling=True)
_scatter64 = _make_edge_scatter(WOUT_PAD, jnp.float32, tc_tiling=False)

_BR = 1024


def _mm_body(x_ref, w_ref, o_ref):
  o_ref[...] = jnp.dot(x_ref[...], w_ref[...],
                       preferred_element_type=jnp.float32
                       ).astype(o_ref.dtype)


def _combine_mm_body(p_ref, b_ref, w_ref, o_ref):
  p = p_ref[0].astype(jnp.float32) + p_ref[1].astype(jnp.float32)
  h = jnp.maximum(p + b_ref[...], 0.0)
  o_ref[...] = jnp.dot(h, w_ref[...], preferred_element_type=jnp.float32
                       ).astype(o_ref.dtype)


def _combine_body(p_ref, b_ref, o_ref):
  p = p_ref[0].astype(jnp.float32) + p_ref[1].astype(jnp.float32)
  o_ref[...] = jnp.maximum(p + b_ref[...], 0.0).astype(o_ref.dtype)


def _final_body(p_ref, b_ref, o_ref):
  v = (p_ref[0].astype(jnp.float32)
       + p_ref[1].astype(jnp.float32)) + b_ref[...]
  col = lax.broadcasted_iota(jnp.int32, v.shape, 1)
  valid = col < N_CLASS
  vm = jnp.where(valid, v, -jnp.inf)
  m = jnp.max(vm, axis=1, keepdims=True)
  ex = jnp.where(valid, jnp.exp(v - m), 0.0)
  lse = jnp.log(jnp.sum(ex, axis=1, keepdims=True)) + m
  o_ref[...] = (v - lse)[:, :N_CLASS]


def _tc_matmul(x, w, out_dtype=jnp.float32):
  n, k = NP, x.shape[1]
  kw, m = w.shape
  return pl.pallas_call(
      _mm_body,
      grid=(n // _BR,),
      in_specs=[pl.BlockSpec((_BR, k), lambda i: (i, 0)),
                pl.BlockSpec((kw, m), lambda i: (0, 0))],
      out_specs=pl.BlockSpec((_BR, m), lambda i: (i, 0)),
      out_shape=jax.ShapeDtypeStruct((n, m), out_dtype),
  )(x, w)


def _tc_combine_mm(p, b, w, out_dtype=jnp.float32):
  _, n, k = p.shape
  kw, m = w.shape
  return pl.pallas_call(
      _combine_mm_body,
      grid=(n // _BR,),
      in_specs=[pl.BlockSpec((NC, _BR, k), lambda i: (0, i, 0)),
                pl.BlockSpec((1, k), lambda i: (0, 0)),
                pl.BlockSpec((kw, m), lambda i: (0, 0))],
      out_specs=pl.BlockSpec((_BR, m), lambda i: (i, 0)),
      out_shape=jax.ShapeDtypeStruct((n, m), out_dtype),
  )(p, b, w)


def _tc_combine(p, b, out_dtype=jnp.float32):
  _, n, k = p.shape
  return pl.pallas_call(
      _combine_body,
      grid=(n // _BR,),
      in_specs=[pl.BlockSpec((NC, _BR, k), lambda i: (0, i, 0)),
                pl.BlockSpec((1, k), lambda i: (0, 0))],
      out_specs=pl.BlockSpec((_BR, k), lambda i: (i, 0)),
      out_shape=jax.ShapeDtypeStruct((n, k), out_dtype),
  )(p, b)


def _tc_final(p, b):
  _, n, m = p.shape
  return pl.pallas_call(
      _final_body,
      grid=(n // _BR,),
      in_specs=[pl.BlockSpec((NC, _BR, m), lambda i: (0, i, 0)),
                pl.BlockSpec((1, m), lambda i: (0, 0))],
      out_specs=pl.BlockSpec((_BR, N_CLASS), lambda i: (i, 0)),
      out_shape=jax.ShapeDtypeStruct((N_NODES, N_CLASS), jnp.float32),
  )(p, b)


def kernel(x, edge_index, W0, b0, W1, b1, Wout, bout):
  edge_flat = edge_index.astype(jnp.int32).reshape(2 * N_EDGES)
  wout_p = jnp.pad(Wout, ((0, 0), (0, WOUT_PAD - N_CLASS)))
  bout_p = jnp.pad(bout, (0, WOUT_PAD - N_CLASS)).reshape(1, WOUT_PAD)
  b0r = b0.reshape(1, D)
  b1r = b1.reshape(1, D)

  support0 = _tc_matmul(x, W0)
  p0 = _scatter128(support0, edge_flat)
  support1 = _tc_combine_mm(p0, b0r, W1)
  p1 = _scatter128(support1, edge_flat)
  support2 = _tc_combine_mm(p1, b1r, wout_p)
  p2 = _scatter64(support2, edge_flat)
  return _tc_final(p2, bout_p)


# speedup vs baseline: 1.0990x; 1.0316x over previous
import functools

import jax
import jax.numpy as jnp
from jax import lax
from jax.experimental import pallas as pl
from jax.experimental.pallas import tpu as pltpu
from jax.experimental.pallas import tpu_sc as plsc

N_NODES = 10000
N_EDGES = 320000
D = 128
WOUT_PAD = 64
N_CLASS = 41

NC = 2
NS = 16
NW = NC * NS

NP = 10240
ROWS_PT = NP // NS
EPT = N_EDGES // NW
CHUNK = 80
NCHUNK = EPT // CHUNK
NBUF = 4
GDEPTH = 2
NIB = 8
UNROLL = 8


def _make_edge_scatter(W, dtype=jnp.float32, tc_tiling=True):
  mesh = plsc.VectorSubcoreMesh(core_axis_name="c", subcore_axis_name="s")

  rows_t = [pltpu.VMEM((CHUNK, W), dtype) for _ in range(NBUF)]
  sidx_t = [pltpu.VMEM((CHUNK,), jnp.int32) for _ in range(NIB)]
  didx_t = [pltpu.VMEM((CHUNK,), jnp.int32) for _ in range(NIB)]
  sem_t = [pltpu.SemaphoreType.DMA for _ in range(2 * NBUF + NIB)]

  @functools.partial(
      pl.kernel,
      mesh=mesh,
      compiler_params=pltpu.CompilerParams(
          use_tc_tiling_on_sc=None if tc_tiling else False),
      out_type=jax.ShapeDtypeStruct((NC, NP, W), dtype),
      scratch_types=[
          pltpu.VMEM_SHARED((NP, W), dtype),
      ] + rows_t + sidx_t + didx_t + sem_t,
  )
  def scatter_kernel(support, edge_flat, out, agg, *bufs):
    rows = bufs[:NBUF]
    sidx = bufs[NBUF:NBUF + NIB]
    didx = bufs[NBUF + NIB:NBUF + 2 * NIB]
    gsem = bufs[NBUF + 2 * NIB:2 * NBUF + 2 * NIB]
    ssem = bufs[2 * NBUF + 2 * NIB:3 * NBUF + 2 * NIB]
    isem = bufs[3 * NBUF + 2 * NIB:3 * NBUF + 3 * NIB]
    cid = lax.axis_index("c")
    sid = lax.axis_index("s")
    wid = cid * NS + sid
    rbase = sid * ROWS_PT

    ebase = wid * EPT

    def i_start(j, b):
      e0 = ebase + j * CHUNK
      pltpu.async_copy(edge_flat.at[pl.ds(e0, CHUNK)], sidx[b], isem[b])
      pltpu.async_copy(edge_flat.at[pl.ds(N_EDGES + e0, CHUNK)], didx[b],
                       isem[b])

    def i_wait(j, b):
      e0 = ebase + j * CHUNK
      pltpu.make_async_copy(edge_flat.at[pl.ds(e0, CHUNK)], sidx[b],
                            isem[b]).wait()
      pltpu.make_async_copy(edge_flat.at[pl.ds(N_EDGES + e0, CHUNK)],
                            didx[b], isem[b]).wait()

    def g_start(j, b, ib):
      pltpu.async_copy(support.at[sidx[ib]], rows[b], gsem[b])

    def g_wait(j, b, ib):
      pltpu.make_async_copy(support.at[sidx[ib]], rows[b], gsem[b]).wait()

    def s_start(j, b, ib):
      pltpu.async_copy(rows[b], agg.at[didx[ib]], ssem[b], add=True)

    def s_wait(j, b, ib):
      pltpu.make_async_copy(rows[b], agg.at[didx[ib]], ssem[b]).wait()

    def slot(j, b, ib, swait, gstart, istart):
      g_wait(j, b, ib)
      s_start(j, b, ib)
      if swait:
        s_wait(j - GDEPTH, (b + GDEPTH) % NBUF, (ib - GDEPTH) % NIB)
      if gstart:
        i_wait(j + GDEPTH, (ib + GDEPTH) % NIB)
        g_start(j + GDEPTH, (b + GDEPTH) % NBUF, (ib + GDEPTH) % NIB)
      if istart:
        i_start(j + GDEPTH + 1, (ib + GDEPTH + 1) % NIB)

    for j in range(GDEPTH + 1):
      i_start(j, j % NIB)
    for j in range(GDEPTH):
      i_wait(j, j % NIB)
      g_start(j, j % NBUF, j % NIB)

    zbufs = list(range(GDEPTH, NBUF))
    if dtype == jnp.float32:
      zvec = jnp.zeros((16,), dtype)

      def zfill_body(r, _):
        for q in range(W // 16):
          for zb in zbufs:
            rows[zb][r, pl.ds(q * 16, 16)] = zvec
        return _
      lax.fori_loop(0, CHUNK, zfill_body, None)
    else:
      zvec = jnp.zeros((2, 16), dtype)

      def zfill_body(i, _):
        for q in range(W // 16):
          for zb in zbufs:
            rows[zb][pl.ds(2 * i, 2), pl.ds(q * 16, 16)] = zvec
        return _
      lax.fori_loop(0, CHUNK // 2, zfill_body, None)
    for i in range(ROWS_PT // CHUNK):
      b = zbufs[i % len(zbufs)]
      pltpu.async_copy(rows[b], agg.at[pl.ds(rbase + i * CHUNK, CHUNK)],
                       ssem[b])
    for i in range(ROWS_PT // CHUNK):
      b = zbufs[i % len(zbufs)]
      pltpu.make_async_copy(rows[b], agg.at[pl.ds(rbase + i * CHUNK, CHUNK)],
                            ssem[b]).wait()
    plsc.subcore_barrier()

    for j in range(UNROLL):
      slot(j, j % NBUF, j % NIB, swait=(j >= GDEPTH), gstart=True,
           istart=True)

    def ring_body(g, _):
      j0 = g * UNROLL
      for k in range(UNROLL):
        slot(j0 + k, k % NBUF, k % NIB, swait=True, gstart=True,
             istart=True)
      return _
    _K = (NCHUNK - UNROLL - 1) // UNROLL
    lax.fori_loop(1, _K, ring_body, None)

    for j in range(_K * UNROLL, NCHUNK):
      slot(j, j % NBUF, j % NIB, swait=True,
           gstart=(j + GDEPTH < NCHUNK),
           istart=(j + GDEPTH + 1 < NCHUNK))
    for j in range(NCHUNK - GDEPTH, NCHUNK):
      s_wait(j, j % NBUF, j % NIB)
    plsc.subcore_barrier()

    pltpu.sync_copy(agg.at[pl.ds(rbase, ROWS_PT)],
                    out.at[cid].at[pl.ds(rbase, ROWS_PT)])

  return scatter_kernel


_scatter128 = _make_edge_scatter(D, jnp.float32, tc_tiling=True)
_scatter64 = _make_edge_scatter(WOUT_PAD, jnp.float32, tc_tiling=False)

_BR = 2048


def _mm_body(x_ref, w_ref, o_ref):
  o_ref[...] = jnp.dot(x_ref[...], w_ref[...],
                       preferred_element_type=jnp.float32
                       ).astype(o_ref.dtype)


def _combine_mm_body(p_ref, b_ref, w_ref, o_ref):
  p = p_ref[0].astype(jnp.float32) + p_ref[1].astype(jnp.float32)
  h = jnp.maximum(p + b_ref[...], 0.0)
  o_ref[...] = jnp.dot(h, w_ref[...], preferred_element_type=jnp.float32
                       ).astype(o_ref.dtype)


def _combine_body(p_ref, b_ref, o_ref):
  p = p_ref[0].astype(jnp.float32) + p_ref[1].astype(jnp.float32)
  o_ref[...] = jnp.maximum(p + b_ref[...], 0.0).astype(o_ref.dtype)


def _final_body(p_ref, b_ref, o_ref):
  v = (p_ref[0].astype(jnp.float32)
       + p_ref[1].astype(jnp.float32)) + b_ref[...]
  col = lax.broadcasted_iota(jnp.int32, v.shape, 1)
  valid = col < N_CLASS
  vm = jnp.where(valid, v, -jnp.inf)
  m = jnp.max(vm, axis=1, keepdims=True)
  ex = jnp.where(valid, jnp.exp(v - m), 0.0)
  lse = jnp.log(jnp.sum(ex, axis=1, keepdims=True)) + m
  o_ref[...] = (v - lse)[:, :N_CLASS]


def _tc_matmul(x, w, out_dtype=jnp.float32):
  n, k = NP, x.shape[1]
  kw, m = w.shape
  return pl.pallas_call(
      _mm_body,
      grid=(n // _BR,),
      in_specs=[pl.BlockSpec((_BR, k), lambda i: (i, 0)),
                pl.BlockSpec((kw, m), lambda i: (0, 0))],
      out_specs=pl.BlockSpec((_BR, m), lambda i: (i, 0)),
      out_shape=jax.ShapeDtypeStruct((n, m), out_dtype),
  )(x, w)


def _tc_combine_mm(p, b, w, out_dtype=jnp.float32):
  _, n, k = p.shape
  kw, m = w.shape
  return pl.pallas_call(
      _combine_mm_body,
      grid=(n // _BR,),
      in_specs=[pl.BlockSpec((NC, _BR, k), lambda i: (0, i, 0)),
                pl.BlockSpec((1, k), lambda i: (0, 0)),
                pl.BlockSpec((kw, m), lambda i: (0, 0))],
      out_specs=pl.BlockSpec((_BR, m), lambda i: (i, 0)),
      out_shape=jax.ShapeDtypeStruct((n, m), out_dtype),
  )(p, b, w)


def _tc_combine(p, b, out_dtype=jnp.float32):
  _, n, k = p.shape
  return pl.pallas_call(
      _combine_body,
      grid=(n // _BR,),
      in_specs=[pl.BlockSpec((NC, _BR, k), lambda i: (0, i, 0)),
                pl.BlockSpec((1, k), lambda i: (0, 0))],
      out_specs=pl.BlockSpec((_BR, k), lambda i: (i, 0)),
      out_shape=jax.ShapeDtypeStruct((n, k), out_dtype),
  )(p, b)


def _tc_final(p, b):
  _, n, m = p.shape
  return pl.pallas_call(
      _final_body,
      grid=(n // _BR,),
      in_specs=[pl.BlockSpec((NC, _BR, m), lambda i: (0, i, 0)),
                pl.BlockSpec((1, m), lambda i: (0, 0))],
      out_specs=pl.BlockSpec((_BR, N_CLASS), lambda i: (i, 0)),
      out_shape=jax.ShapeDtypeStruct((N_NODES, N_CLASS), jnp.float32),
  )(p, b)


def kernel(x, edge_index, W0, b0, W1, b1, Wout, bout):
  edge_flat = edge_index.astype(jnp.int32).reshape(2 * N_EDGES)
  wout_p = jnp.pad(Wout, ((0, 0), (0, WOUT_PAD - N_CLASS)))
  bout_p = jnp.pad(bout, (0, WOUT_PAD - N_CLASS)).reshape(1, WOUT_PAD)
  b0r = b0.reshape(1, D)
  b1r = b1.reshape(1, D)

  support0 = _tc_matmul(x, W0)
  p0 = _scatter128(support0, edge_flat)
  support1 = _tc_combine_mm(p0, b0r, W1)
  p1 = _scatter128(support1, edge_flat)
  support2 = _tc_combine_mm(p1, b1r, wout_p)
  p2 = _scatter64(support2, edge_flat)
  return _tc_final(p2, bout_p)
